# baseline (speedup 1.0000x reference)
import jax
import jax.numpy as jnp
from jax.experimental import pallas as pl

NUM_HEADS = 8
D_HEAD = 16


def _ident(x_ref, o_ref):
    o_ref[...] = x_ref[...]


def kernel(feature, fc_weight, attn_weight, edge_index, etype):
    src = edge_index[0]
    dst = edge_index[1]
    n_nodes = feature.shape[0]
    z_all = jnp.einsum('nd,rdo->nro', feature, fc_weight)
    z_s = z_all[src, etype].reshape(-1, NUM_HEADS, D_HEAD)
    z_d = z_all[dst, etype].reshape(-1, NUM_HEADS, D_HEAD)
    z2 = jnp.concatenate([z_s, z_d], axis=2)
    a = jnp.sum(z2 * attn_weight[etype], axis=2)
    e = jax.nn.leaky_relu(a, negative_slope=0.01)
    e_max = jax.ops.segment_max(e, dst, num_segments=n_nodes)
    e_exp = jnp.exp(e - e_max[dst])
    denom = jax.ops.segment_sum(e_exp, dst, num_segments=n_nodes)
    alpha = e_exp / denom[dst]
    h = jax.ops.segment_sum(alpha[:, :, None] * z_s, dst, num_segments=n_nodes)
    h2 = h.reshape(n_nodes, NUM_HEADS * D_HEAD)
    out = pl.pallas_call(
        _ident,
        out_shape=jax.ShapeDtypeStruct(h2.shape, h2.dtype),
    )(h2)
    return out.reshape(n_nodes, NUM_HEADS, D_HEAD)


# speedup vs baseline: 8.1582x; 8.1582x over previous
import functools

import jax
import jax.numpy as jnp
from jax import lax
from jax.experimental import pallas as pl
from jax.experimental.pallas import tpu as pltpu
from jax.experimental.pallas import tpu_sc as plsc

N = 10000
E = 160000
IN_DIM = 128
OUT_DIM = 128
R = 8
H = 8
DH = 16

NC = 2
NS = 16
NW = NC * NS
RNG = 313
W = 640
NWIN = E // W
SUB = 128
NBLK = 25
BLK = N // NBLK


def _tc_body(x_ref, fc_ref, wsd_ref, z_ref, asd_ref):
    x = x_ref[...]
    for r in range(R):
        zr = lax.dot_general(x, fc_ref[r], (((1,), (0,)), ((), ())),
                             preferred_element_type=jnp.float32,
                             precision=lax.Precision.HIGHEST)
        z_ref[:, r, :] = zr
        asd_ref[:, r, :] = lax.dot_general(zr, wsd_ref[r],
                                           (((1,), (0,)), ((), ())),
                                           preferred_element_type=jnp.float32,
                                           precision=lax.Precision.HIGHEST)


def _sc_body(z_hbm, asd_hbm, src_hbm, dst_hbm, et_hbm, out_hbm,
             dstw, srcw, etw, csrow, cdrow, cdl,
             sbuf, dbuf, wbuf, zbuf, m_ref, s_ref, h_ref, sem):
    i32 = jnp.int32
    f32 = jnp.float32
    wid = lax.axis_index("s") * NC + lax.axis_index("c")
    lo = wid * RNG
    iota = lax.iota(i32, 16)
    h8 = iota & 7
    kk01 = iota >> 3
    kk10 = 1 - kk01

    def init_ms(i, _):
        m_ref[pl.ds(i * 16, 16)] = jnp.full((16,), -3.0e38, f32)
        s_ref[pl.ds(i * 16, 16)] = jnp.zeros((16,), f32)
        return 0
    lax.fori_loop(0, (RNG * H + 16) // 16, init_ms, 0)

    def init_h(i, _):
        h_ref[pl.ds(i * 16, 16)] = jnp.zeros((16,), f32)
        return 0
    lax.fori_loop(0, (RNG * 128 + 128) // 16, init_h, 0)

    def init_idx(i, _):
        v = i * 16 + iota
        csrow[pl.ds(i * 16, 16)] = v
        cdrow[pl.ds(i * 16, 16)] = v
        cdl[pl.ds(i * 16, 16)] = jnp.full((16,), RNG, i32)
        return 0
    lax.fori_loop(0, (W + 16) // 16, init_idx, 0)

    def compress_window(w):
        pltpu.sync_copy(dst_hbm.at[pl.ds(w * W, W)], dstw)
        pltpu.sync_copy(src_hbm.at[pl.ds(w * W, W)], srcw)
        pltpu.sync_copy(et_hbm.at[pl.ds(w * W, W)], etw)

        lov = jnp.full((16,), lo, i32)
        hiv = jnp.full((16,), lo + RNG, i32)

        def chunk(c, cnt):
            d16 = dstw[pl.ds(c * 16, 16)]
            s16 = srcw[pl.ds(c * 16, 16)]
            t16 = etw[pl.ds(c * 16, 16)]
            mask = (d16 >= lov) & (d16 < hiv)
            mi = mask.astype(i32)
            pos = jnp.full((16,), cnt - 1, i32) + jnp.cumsum(mi)
            plsc.store_scatter(csrow, [pos], s16 * R + t16, mask=mask)
            plsc.store_scatter(cdrow, [pos], d16 * R + t16, mask=mask)
            plsc.store_scatter(cdl, [pos], d16 - lov, mask=mask)
            return cnt + jnp.sum(mi)

        cnt = lax.fori_loop(0, W // 16, chunk, jnp.int32(0))
        cntv = jnp.full((16,), cnt, i32)
        lane0 = iota < 1
        plsc.store_scatter(csrow, [cntv], jnp.full((16,), wid * R, i32), mask=lane0)
        plsc.store_scatter(cdrow, [cntv], jnp.full((16,), wid * R, i32), mask=lane0)
        plsc.store_scatter(cdl, [cntv], jnp.full((16,), RNG, i32), mask=lane0)
        return cnt

    def logits_for_pair(g, k2):
        base = k2 * 2
        rows = base + kk01
        rows_sw = base + kk10
        asv = plsc.load_gather(sbuf, [rows, h8])
        adv = plsc.load_gather(dbuf, [rows, h8 + 8])
        av = asv + adv
        ev = jnp.maximum(av, 0.01 * av)
        as_sw = plsc.load_gather(sbuf, [rows_sw, h8])
        ad_sw = plsc.load_gather(dbuf, [rows_sw, h8 + 8])
        aw = as_sw + ad_sw
        esw = jnp.maximum(aw, 0.01 * aw)
        ge = g * SUB + base
        dlv = plsc.load_gather(cdl, [ge + kk01])
        dsw = plsc.load_gather(cdl, [ge + kk10])
        eq = dlv == dsw
        midx = dlv * H + h8
        return ev, esw, eq, midx

    def p1_window(w, _):
        cnt = compress_window(w)
        cnte = cnt + (cnt & 1)
        pairs = cnte // 2

        def gbatch(g, _):
            pltpu.async_copy(asd_hbm.at[csrow.at[pl.ds(g * SUB, SUB)]], sbuf, sem).wait()
            pltpu.async_copy(asd_hbm.at[cdrow.at[pl.ds(g * SUB, SUB)]], dbuf, sem).wait()

            def pair(k2, _):
                ev, esw, eq, midx = logits_for_pair(g, k2)
                ecomb = jnp.where(eq, jnp.maximum(ev, esw), ev)
                mg = plsc.load_gather(m_ref, [midx])
                plsc.store_scatter(m_ref, [midx], jnp.maximum(mg, ecomb))
                return 0

            lax.fori_loop(0, jnp.minimum(SUB // 2, pairs - g * (SUB // 2)), pair, 0)
            return 0

        lax.fori_loop(0, (cnte + SUB - 1) // SUB, gbatch, 0)
        return 0

    lax.fori_loop(0, NWIN, p1_window, 0)

    def p2_window(w, _):
        cnt = compress_window(w)
        cnte = cnt + (cnt & 1)
        pairs = cnte // 2

        def gbatch(g, _):
            pltpu.async_copy(asd_hbm.at[csrow.at[pl.ds(g * SUB, SUB)]], sbuf, sem).wait()
            pltpu.async_copy(asd_hbm.at[cdrow.at[pl.ds(g * SUB, SUB)]], dbuf, sem).wait()

            def pair(k2, _):
                ev, esw, eq, midx = logits_for_pair(g, k2)
                mrow = plsc.load_gather(m_ref, [midx])
                wv = jnp.exp(ev - mrow)
                wsw = jnp.exp(esw - mrow)
                wcomb = jnp.where(eq, wv + wsw, wv)
                sg = plsc.load_gather(s_ref, [midx])
                plsc.store_scatter(s_ref, [midx], sg + wcomb)
                wbuf[pl.ds((g * (SUB // 2) + k2) * 16, 16)] = wv
                return 0

            lax.fori_loop(0, jnp.minimum(SUB // 2, pairs - g * (SUB // 2)), pair, 0)
            return 0

        lax.fori_loop(0, (cnte + SUB - 1) // SUB, gbatch, 0)

        def zbatch(g, _):
            pltpu.async_copy(z_hbm.at[csrow.at[pl.ds(g * SUB, SUB)]], zbuf, sem).wait()

            def edge(k, _):
                ge = g * SUB + k
                dlv = plsc.load_gather(cdl, [jnp.full((16,), ge, i32)])
                kv = jnp.full((16,), k, i32)
                hbase = dlv * 128 + iota
                for j in range(H):
                    wj = plsc.load_gather(wbuf, [jnp.full((16,), ge * H + j, i32)])
                    zv = plsc.load_gather(zbuf, [kv, j * 16 + iota])
                    hidx = hbase + j * 16
                    hg = plsc.load_gather(h_ref, [hidx])
                    plsc.store_scatter(h_ref, [hidx], hg + wj * zv)
                return 0

            lax.fori_loop(0, jnp.minimum(SUB, cnte - g * SUB), edge, 0)
            return 0

        lax.fori_loop(0, (cnte + SUB - 1) // SUB, zbatch, 0)
        return 0

    lax.fori_loop(0, NWIN, p2_window, 0)

    def node(n, _):
        for j in range(H):
            sb = plsc.load_gather(s_ref, [jnp.full((16,), n * H + j, i32)])
            idx = n * 128 + j * 16 + iota
            hseg = plsc.load_gather(h_ref, [idx])
            plsc.store_scatter(h_ref, [idx],
                               jnp.where(sb > 0, hseg / sb, jnp.zeros((16,), f32)))
        return 0

    lax.fori_loop(0, RNG, node, 0)
    pltpu.sync_copy(h_ref.at[pl.ds(0, RNG * 128)],
                    out_hbm.at[pl.ds(lo * 128, RNG * 128)])


@functools.partial(
    pl.kernel,
    mesh=plsc.VectorSubcoreMesh(core_axis_name="c", subcore_axis_name="s"),
    compiler_params=pltpu.CompilerParams(needs_layout_passes=False,
                                         use_tc_tiling_on_sc=False),
    out_type=jax.ShapeDtypeStruct((NW * RNG * 128,), jnp.float32),
    scratch_types=[
        pltpu.VMEM((W,), jnp.int32),
        pltpu.VMEM((W,), jnp.int32),
        pltpu.VMEM((W,), jnp.int32),
        pltpu.VMEM((W + 16,), jnp.int32),
        pltpu.VMEM((W + 16,), jnp.int32),
        pltpu.VMEM((W + 16,), jnp.int32),
        pltpu.VMEM((SUB, 16), jnp.float32),
        pltpu.VMEM((SUB, 16), jnp.float32),
        pltpu.VMEM((W * H,), jnp.float32),
        pltpu.VMEM((SUB, 128), jnp.float32),
        pltpu.VMEM((RNG * H + 16,), jnp.float32),
        pltpu.VMEM((RNG * H + 16,), jnp.float32),
        pltpu.VMEM((RNG * 128 + 128,), jnp.float32),
        pltpu.SemaphoreType.DMA,
    ],
)
def _sc_kernel(z_hbm, asd_hbm, src_hbm, dst_hbm, et_hbm, out_hbm, *scratch):
    _sc_body(z_hbm, asd_hbm, src_hbm, dst_hbm, et_hbm, out_hbm, *scratch)


def kernel(feature, fc_weight, attn_weight, edge_index, etype):
    A = attn_weight.reshape(R, H, 2, DH)
    eye = jnp.eye(H, dtype=jnp.float32)
    Ws = jnp.einsum('rhk,hj->rhkj', A[:, :, 0, :], eye).reshape(R, OUT_DIM, H)
    Wd = jnp.einsum('rhk,hj->rhkj', A[:, :, 1, :], eye).reshape(R, OUT_DIM, H)
    wsd = jnp.concatenate([Ws, Wd], axis=2)

    z_all, asd = pl.pallas_call(
        _tc_body,
        grid=(NBLK,),
        in_specs=[
            pl.BlockSpec((BLK, IN_DIM), lambda i: (i, 0)),
            pl.BlockSpec((R, IN_DIM, OUT_DIM), lambda i: (0, 0, 0)),
            pl.BlockSpec((R, OUT_DIM, 16), lambda i: (0, 0, 0)),
        ],
        out_specs=[
            pl.BlockSpec((BLK, R, OUT_DIM), lambda i: (i, 0, 0)),
            pl.BlockSpec((BLK, R, 16), lambda i: (i, 0, 0)),
        ],
        out_shape=[
            jax.ShapeDtypeStruct((N, R, OUT_DIM), jnp.float32),
            jax.ShapeDtypeStruct((N, R, 16), jnp.float32),
        ],
    )(feature, fc_weight, wsd)

    z_flat = z_all.reshape(N * R, OUT_DIM)
    asd_flat = asd.reshape(N * R, 16)
    src = edge_index[0]
    dst = edge_index[1]
    out1d = _sc_kernel(z_flat, asd_flat, src, dst, etype)
    return out1d[:N * 128].reshape(N, H, DH)


# speedup vs baseline: 15.7801x; 1.9343x over previous
import functools

import jax
import jax.numpy as jnp
from jax import lax
from jax.experimental import pallas as pl
from jax.experimental.pallas import tpu as pltpu
from jax.experimental.pallas import tpu_sc as plsc

N = 10000
E = 160000
IN_DIM = 128
OUT_DIM = 128
R = 8
H = 8
DH = 16

NC = 2
NS = 16
NW = NC * NS
RNG = 313
W = 2000
NWIN = E // W
SUB = 128
NBLK = 25
BLK = N // NBLK


def _tc_body(x_ref, fc_ref, wsd_ref, z_ref, asd_ref):
    x = x_ref[...]
    for r in range(R):
        zr = lax.dot_general(x, fc_ref[r], (((1,), (0,)), ((), ())),
                             preferred_element_type=jnp.float32,
                             precision=lax.Precision.HIGHEST)
        z_ref[:, r, :] = zr
        asd_ref[:, r, :] = lax.dot_general(zr, wsd_ref[r],
                                           (((1,), (0,)), ((), ())),
                                           preferred_element_type=jnp.float32,
                                           precision=lax.Precision.HIGHEST)


def _sc_body(z_hbm, asd_hbm, src_hbm, dst_hbm, et_hbm, out_hbm,
             dstw, srcw, etw, csrow, cdrow, cdl,
             sbuf, dbuf, wbuf, zbuf, m_ref, s_ref, h_ref, sem):
    i32 = jnp.int32
    f32 = jnp.float32
    wid = lax.axis_index("s") * NC + lax.axis_index("c")
    lo = wid * RNG
    iota = lax.iota(i32, 16)
    h8 = iota & 7
    kk01 = iota >> 3
    kk10 = 1 - kk01

    def init_ms(i, _):
        m_ref[pl.ds(i * 16, 16)] = jnp.full((16,), -3.0e38, f32)
        s_ref[pl.ds(i * 16, 16)] = jnp.zeros((16,), f32)
        return 0
    lax.fori_loop(0, (RNG * H + 16) // 16, init_ms, 0)

    def init_h(i, _):
        h_ref[pl.ds(i * 16, 16)] = jnp.zeros((16,), f32)
        return 0
    lax.fori_loop(0, (RNG * 128 + 128) // 16, init_h, 0)

    def init_idx(i, _):
        v = i * 16 + iota
        csrow[pl.ds(i * 16, 16)] = v
        cdrow[pl.ds(i * 16, 16)] = v
        cdl[pl.ds(i * 16, 16)] = jnp.full((16,), RNG, i32)
        return 0
    lax.fori_loop(0, (W + 48) // 16, init_idx, 0)

    def compress_window(w):
        pltpu.sync_copy(dst_hbm.at[pl.ds(w * W, W)], dstw)
        pltpu.sync_copy(src_hbm.at[pl.ds(w * W, W)], srcw)
        pltpu.sync_copy(et_hbm.at[pl.ds(w * W, W)], etw)

        lov = jnp.full((16,), lo, i32)
        hiv = jnp.full((16,), lo + RNG, i32)

        def chunk(c, cnt):
            d16 = dstw[pl.ds(c * 16, 16)]
            s16 = srcw[pl.ds(c * 16, 16)]
            t16 = etw[pl.ds(c * 16, 16)]
            mask = (d16 >= lov) & (d16 < hiv)
            mi = mask.astype(i32)
            pos = jnp.full((16,), cnt - 1, i32) + jnp.cumsum(mi)
            plsc.store_scatter(csrow, [pos], s16 * R + t16, mask=mask)
            plsc.store_scatter(cdrow, [pos], d16 * R + t16, mask=mask)
            plsc.store_scatter(cdl, [pos], d16 - lov, mask=mask)
            return cnt + jnp.sum(mi)

        cnt = lax.fori_loop(0, W // 16, chunk, jnp.int32(0))
        cntv = jnp.full((16,), cnt, i32)
        lane0 = iota < 1
        plsc.store_scatter(csrow, [cntv], jnp.full((16,), wid * R, i32), mask=lane0)
        plsc.store_scatter(cdrow, [cntv], jnp.full((16,), wid * R, i32), mask=lane0)
        plsc.store_scatter(cdl, [cntv], jnp.full((16,), RNG, i32), mask=lane0)
        return cnt

    def logits_for_pair(g, k2):
        base = k2 * 2
        rows = base + kk01
        rows_sw = base + kk10
        asv = plsc.load_gather(sbuf, [rows, h8])
        adv = plsc.load_gather(dbuf, [rows, h8 + 8])
        av = asv + adv
        ev = jnp.maximum(av, 0.01 * av)
        as_sw = plsc.load_gather(sbuf, [rows_sw, h8])
        ad_sw = plsc.load_gather(dbuf, [rows_sw, h8 + 8])
        aw = as_sw + ad_sw
        esw = jnp.maximum(aw, 0.01 * aw)
        ge = g * SUB + base
        dlv = plsc.load_gather(cdl, [ge + kk01])
        dsw = plsc.load_gather(cdl, [ge + kk10])
        eq = dlv == dsw
        midx = dlv * H + h8
        return ev, esw, eq, midx

    def p1_window(w, _):
        cnt = compress_window(w)
        cnte = cnt + (cnt & 1)
        pairs = cnte // 2

        def gbatch(g, _):
            c1 = pltpu.async_copy(asd_hbm.at[csrow.at[pl.ds(g * SUB, SUB)]], sbuf, sem)
            c2 = pltpu.async_copy(asd_hbm.at[cdrow.at[pl.ds(g * SUB, SUB)]], dbuf, sem)
            c1.wait()
            c2.wait()

            def pair(k2, _):
                ev, esw, eq, midx = logits_for_pair(g, k2)
                ecomb = jnp.where(eq, jnp.maximum(ev, esw), ev)
                mg = plsc.load_gather(m_ref, [midx])
                plsc.store_scatter(m_ref, [midx], jnp.maximum(mg, ecomb))
                return 0

            lax.fori_loop(0, jnp.minimum(SUB // 2, pairs - g * (SUB // 2)), pair, 0)
            return 0

        lax.fori_loop(0, (cnte + SUB - 1) // SUB, gbatch, 0)
        return 0

    lax.fori_loop(0, NWIN, p1_window, 0)

    def p2_window(w, _):
        cnt = compress_window(w)
        cnte = cnt + (cnt & 1)
        pairs = cnte // 2

        def gbatch(g, _):
            c1 = pltpu.async_copy(asd_hbm.at[csrow.at[pl.ds(g * SUB, SUB)]], sbuf, sem)
            c2 = pltpu.async_copy(asd_hbm.at[cdrow.at[pl.ds(g * SUB, SUB)]], dbuf, sem)
            c1.wait()
            c2.wait()

            def pair(k2, _):
                ev, esw, eq, midx = logits_for_pair(g, k2)
                mrow = plsc.load_gather(m_ref, [midx])
                wv = jnp.exp(ev - mrow)
                wsw = jnp.exp(esw - mrow)
                wcomb = jnp.where(eq, wv + wsw, wv)
                sg = plsc.load_gather(s_ref, [midx])
                plsc.store_scatter(s_ref, [midx], sg + wcomb)
                wbuf[pl.ds((g * (SUB // 2) + k2) * 16, 16)] = wv
                return 0

            lax.fori_loop(0, jnp.minimum(SUB // 2, pairs - g * (SUB // 2)), pair, 0)
            return 0

        lax.fori_loop(0, (cnte + SUB - 1) // SUB, gbatch, 0)

        def zbatch(g, _):
            pltpu.async_copy(z_hbm.at[csrow.at[pl.ds(g * SUB, SUB)]], zbuf, sem).wait()

            def edge(k, _):
                ge = g * SUB + k
                dlv = plsc.load_gather(cdl, [jnp.full((16,), ge, i32)])
                kv = jnp.full((16,), k, i32)
                hbase = dlv * 128 + iota
                for j in range(H):
                    wj = plsc.load_gather(wbuf, [jnp.full((16,), ge * H + j, i32)])
                    zv = plsc.load_gather(zbuf, [kv, j * 16 + iota])
                    hidx = hbase + j * 16
                    hg = plsc.load_gather(h_ref, [hidx])
                    plsc.store_scatter(h_ref, [hidx], hg + wj * zv)
                return 0

            lax.fori_loop(0, jnp.minimum(SUB, cnte - g * SUB), edge, 0)
            return 0

        lax.fori_loop(0, (cnte + SUB - 1) // SUB, zbatch, 0)
        return 0

    lax.fori_loop(0, NWIN, p2_window, 0)

    def node(n, _):
        for j in range(H):
            sb = plsc.load_gather(s_ref, [jnp.full((16,), n * H + j, i32)])
            idx = n * 128 + j * 16 + iota
            hseg = plsc.load_gather(h_ref, [idx])
            plsc.store_scatter(h_ref, [idx],
                               jnp.where(sb > 0, hseg / sb, jnp.zeros((16,), f32)))
        return 0

    lax.fori_loop(0, RNG, node, 0)
    pltpu.sync_copy(h_ref.at[pl.ds(0, RNG * 128)],
                    out_hbm.at[pl.ds(lo * 128, RNG * 128)])


@functools.partial(
    pl.kernel,
    mesh=plsc.VectorSubcoreMesh(core_axis_name="c", subcore_axis_name="s"),
    compiler_params=pltpu.CompilerParams(needs_layout_passes=False,
                                         use_tc_tiling_on_sc=False),
    out_type=jax.ShapeDtypeStruct((NW * RNG * 128,), jnp.float32),
    scratch_types=[
        pltpu.VMEM((W,), jnp.int32),
        pltpu.VMEM((W,), jnp.int32),
        pltpu.VMEM((W,), jnp.int32),
        pltpu.VMEM((W + 48,), jnp.int32),
        pltpu.VMEM((W + 48,), jnp.int32),
        pltpu.VMEM((W + 48,), jnp.int32),
        pltpu.VMEM((SUB, 16), jnp.float32),
        pltpu.VMEM((SUB, 16), jnp.float32),
        pltpu.VMEM((W * H,), jnp.float32),
        pltpu.VMEM((SUB, 128), jnp.float32),
        pltpu.VMEM((RNG * H + 16,), jnp.float32),
        pltpu.VMEM((RNG * H + 16,), jnp.float32),
        pltpu.VMEM((RNG * 128 + 128,), jnp.float32),
        pltpu.SemaphoreType.DMA,
    ],
)
def _sc_kernel(z_hbm, asd_hbm, src_hbm, dst_hbm, et_hbm, out_hbm, *scratch):
    _sc_body(z_hbm, asd_hbm, src_hbm, dst_hbm, et_hbm, out_hbm, *scratch)


def kernel(feature, fc_weight, attn_weight, edge_index, etype):
    A = attn_weight.reshape(R, H, 2, DH)
    eye = jnp.eye(H, dtype=jnp.float32)
    Ws = jnp.einsum('rhk,hj->rhkj', A[:, :, 0, :], eye).reshape(R, OUT_DIM, H)
    Wd = jnp.einsum('rhk,hj->rhkj', A[:, :, 1, :], eye).reshape(R, OUT_DIM, H)
    wsd = jnp.concatenate([Ws, Wd], axis=2)

    z_all, asd = pl.pallas_call(
        _tc_body,
        grid=(NBLK,),
        in_specs=[
            pl.BlockSpec((BLK, IN_DIM), lambda i: (i, 0)),
            pl.BlockSpec((R, IN_DIM, OUT_DIM), lambda i: (0, 0, 0)),
            pl.BlockSpec((R, OUT_DIM, 16), lambda i: (0, 0, 0)),
        ],
        out_specs=[
            pl.BlockSpec((BLK, R, OUT_DIM), lambda i: (i, 0, 0)),
            pl.BlockSpec((BLK, R, 16), lambda i: (i, 0, 0)),
        ],
        out_shape=[
            jax.ShapeDtypeStruct((N, R, OUT_DIM), jnp.float32),
            jax.ShapeDtypeStruct((N, R, 16), jnp.float32),
        ],
    )(feature, fc_weight, wsd)

    z_flat = z_all.reshape(N * R, OUT_DIM)
    asd_flat = asd.reshape(N * R, 16)
    src = edge_index[0]
    dst = edge_index[1]
    out1d = _sc_kernel(z_flat, asd_flat, src, dst, etype)
    return out1d[:N * 128].reshape(N, H, DH)


# speedup vs baseline: 15.9828x; 1.0128x over previous
import functools

import jax
import jax.numpy as jnp
from jax import lax
from jax.experimental import pallas as pl
from jax.experimental.pallas import tpu as pltpu
from jax.experimental.pallas import tpu_sc as plsc

N = 10000
E = 160000
IN_DIM = 128
OUT_DIM = 128
R = 8
H = 8
DH = 16

NC = 2
NS = 16
NW = NC * NS
RNG = 313
W = 2000
NWIN = E // W
SUB = 128
SUBA = 512
NBLK = 25
BLK = N // NBLK


def _tc_body(x_ref, fc_ref, wsd_ref, z_ref, asd_ref):
    x = x_ref[...]
    for r in range(R):
        zr = lax.dot_general(x, fc_ref[r], (((1,), (0,)), ((), ())),
                             preferred_element_type=jnp.float32,
                             precision=lax.Precision.HIGHEST)
        z_ref[:, r, :] = zr
        asd_ref[:, r, :] = lax.dot_general(zr, wsd_ref[r],
                                           (((1,), (0,)), ((), ())),
                                           preferred_element_type=jnp.float32,
                                           precision=lax.Precision.HIGHEST)


def _sc_body(z_hbm, asd_hbm, src_hbm, dst_hbm, et_hbm, out_hbm,
             dstw, srcw, etw, csrow, cdrow, cdl,
             sbuf, dbuf, wbuf, zbuf, m_ref, s_ref, h_ref, sem):
    i32 = jnp.int32
    f32 = jnp.float32
    wid = lax.axis_index("s") * NC + lax.axis_index("c")
    lo = wid * RNG
    iota = lax.iota(i32, 16)
    h8 = iota & 7
    kk01 = iota >> 3
    kk10 = 1 - kk01

    def init_ms(i, _):
        m_ref[pl.ds(i * 16, 16)] = jnp.full((16,), -3.0e38, f32)
        s_ref[pl.ds(i * 16, 16)] = jnp.zeros((16,), f32)
        return 0
    lax.fori_loop(0, (RNG * H + 16) // 16, init_ms, 0)

    def init_h(i, _):
        h_ref[pl.ds(i * 16, 16)] = jnp.zeros((16,), f32)
        return 0
    lax.fori_loop(0, (RNG * 128 + 128) // 16, init_h, 0)

    def init_idx(i, _):
        v = i * 16 + iota
        csrow[pl.ds(i * 16, 16)] = v
        cdrow[pl.ds(i * 16, 16)] = v
        cdl[pl.ds(i * 16, 16)] = jnp.full((16,), RNG, i32)
        return 0
    lax.fori_loop(0, (W + 48) // 16, init_idx, 0)

    def compress_window(w):
        c1 = pltpu.async_copy(dst_hbm.at[pl.ds(w * W, W)], dstw, sem)
        c2 = pltpu.async_copy(src_hbm.at[pl.ds(w * W, W)], srcw, sem)
        c3 = pltpu.async_copy(et_hbm.at[pl.ds(w * W, W)], etw, sem)
        c1.wait()
        c2.wait()
        c3.wait()

        lov = jnp.full((16,), lo, i32)
        hiv = jnp.full((16,), lo + RNG, i32)

        def chunk(c, cnt):
            d16 = dstw[pl.ds(c * 16, 16)]
            s16 = srcw[pl.ds(c * 16, 16)]
            t16 = etw[pl.ds(c * 16, 16)]
            mask = (d16 >= lov) & (d16 < hiv)
            mi = mask.astype(i32)
            pos = jnp.full((16,), cnt - 1, i32) + jnp.cumsum(mi)
            plsc.store_scatter(csrow, [pos], s16 * R + t16, mask=mask)
            plsc.store_scatter(cdrow, [pos], d16 * R + t16, mask=mask)
            plsc.store_scatter(cdl, [pos], d16 - lov, mask=mask)
            return cnt + jnp.sum(mi)

        cnt = lax.fori_loop(0, W // 16, chunk, jnp.int32(0))
        cntv = jnp.full((16,), cnt, i32)
        lane0 = iota < 1
        plsc.store_scatter(csrow, [cntv], jnp.full((16,), wid * R, i32), mask=lane0)
        plsc.store_scatter(cdrow, [cntv], jnp.full((16,), wid * R, i32), mask=lane0)
        plsc.store_scatter(cdl, [cntv], jnp.full((16,), RNG, i32), mask=lane0)
        return cnt

    def logits_for_pair(g, k2):
        base = k2 * 2
        rows = base + kk01
        rows_sw = base + kk10
        asv = plsc.load_gather(sbuf, [rows, h8])
        adv = plsc.load_gather(dbuf, [rows, h8 + 8])
        av = asv + adv
        ev = jnp.maximum(av, 0.01 * av)
        as_sw = plsc.load_gather(sbuf, [rows_sw, h8])
        ad_sw = plsc.load_gather(dbuf, [rows_sw, h8 + 8])
        aw = as_sw + ad_sw
        esw = jnp.maximum(aw, 0.01 * aw)
        ge = g * SUBA + base
        dlv = plsc.load_gather(cdl, [ge + kk01])
        dsw = plsc.load_gather(cdl, [ge + kk10])
        eq = dlv == dsw
        midx = dlv * H + h8
        return ev, esw, eq, midx

    def p1_window(w, _):
        cnt = compress_window(w)
        cnte = cnt + (cnt & 1)
        pairs = cnte // 2

        def gbatch(g, _):
            c1 = pltpu.async_copy(asd_hbm.at[csrow.at[pl.ds(g * SUBA, SUBA)]], sbuf, sem)
            c2 = pltpu.async_copy(asd_hbm.at[cdrow.at[pl.ds(g * SUBA, SUBA)]], dbuf, sem)
            c1.wait()
            c2.wait()

            def pair(k2, _):
                ev, esw, eq, midx = logits_for_pair(g, k2)
                ecomb = jnp.where(eq, jnp.maximum(ev, esw), ev)
                mg = plsc.load_gather(m_ref, [midx])
                plsc.store_scatter(m_ref, [midx], jnp.maximum(mg, ecomb))
                return 0

            lax.fori_loop(0, jnp.minimum(SUBA // 2, pairs - g * (SUBA // 2)), pair, 0)
            return 0

        lax.fori_loop(0, (cnte + SUBA - 1) // SUBA, gbatch, 0)
        return 0

    lax.fori_loop(0, NWIN, p1_window, 0)

    def p2_window(w, _):
        cnt = compress_window(w)
        cnte = cnt + (cnt & 1)
        pairs = cnte // 2

        def gbatch(g, _):
            c1 = pltpu.async_copy(asd_hbm.at[csrow.at[pl.ds(g * SUBA, SUBA)]], sbuf, sem)
            c2 = pltpu.async_copy(asd_hbm.at[cdrow.at[pl.ds(g * SUBA, SUBA)]], dbuf, sem)
            c1.wait()
            c2.wait()

            def pair(k2, _):
                ev, esw, eq, midx = logits_for_pair(g, k2)
                mrow = plsc.load_gather(m_ref, [midx])
                wv = jnp.exp(ev - mrow)
                wsw = jnp.exp(esw - mrow)
                wcomb = jnp.where(eq, wv + wsw, wv)
                sg = plsc.load_gather(s_ref, [midx])
                plsc.store_scatter(s_ref, [midx], sg + wcomb)
                wbuf[pl.ds((g * (SUBA // 2) + k2) * 16, 16)] = wv
                return 0

            lax.fori_loop(0, jnp.minimum(SUBA // 2, pairs - g * (SUBA // 2)), pair, 0)
            return 0

        lax.fori_loop(0, (cnte + SUBA - 1) // SUBA, gbatch, 0)

        def zbatch(g, _):
            pltpu.async_copy(z_hbm.at[csrow.at[pl.ds(g * SUB, SUB)]], zbuf, sem).wait()

            def edge(k, _):
                ge = g * SUB + k
                dlv = plsc.load_gather(cdl, [jnp.full((16,), ge, i32)])
                kv = jnp.full((16,), k, i32)
                hbase = dlv * 128 + iota
                for j in range(H):
                    wj = plsc.load_gather(wbuf, [jnp.full((16,), ge * H + j, i32)])
                    zv = plsc.load_gather(zbuf, [kv, j * 16 + iota])
                    hidx = hbase + j * 16
                    hg = plsc.load_gather(h_ref, [hidx])
                    plsc.store_scatter(h_ref, [hidx], hg + wj * zv)
                return 0

            lax.fori_loop(0, jnp.minimum(SUB, cnte - g * SUB), edge, 0)
            return 0

        lax.fori_loop(0, (cnte + SUB - 1) // SUB, zbatch, 0)
        return 0

    lax.fori_loop(0, NWIN, p2_window, 0)

    def node(n, _):
        for j in range(H):
            sb = plsc.load_gather(s_ref, [jnp.full((16,), n * H + j, i32)])
            idx = n * 128 + j * 16 + iota
            hseg = plsc.load_gather(h_ref, [idx])
            plsc.store_scatter(h_ref, [idx],
                               jnp.where(sb > 0, hseg / sb, jnp.zeros((16,), f32)))
        return 0

    lax.fori_loop(0, RNG, node, 0)
    pltpu.sync_copy(h_ref.at[pl.ds(0, RNG * 128)],
                    out_hbm.at[pl.ds(lo * 128, RNG * 128)])


@functools.partial(
    pl.kernel,
    mesh=plsc.VectorSubcoreMesh(core_axis_name="c", subcore_axis_name="s"),
    compiler_params=pltpu.CompilerParams(needs_layout_passes=False,
                                         use_tc_tiling_on_sc=False),
    out_type=jax.ShapeDtypeStruct((NW * RNG * 128,), jnp.float32),
    scratch_types=[
        pltpu.VMEM((W,), jnp.int32),
        pltpu.VMEM((W,), jnp.int32),
        pltpu.VMEM((W,), jnp.int32),
        pltpu.VMEM((W + 48,), jnp.int32),
        pltpu.VMEM((W + 48,), jnp.int32),
        pltpu.VMEM((W + 48,), jnp.int32),
        pltpu.VMEM((SUBA, 16), jnp.float32),
        pltpu.VMEM((SUBA, 16), jnp.float32),
        pltpu.VMEM((W * H,), jnp.float32),
        pltpu.VMEM((SUB, 128), jnp.float32),
        pltpu.VMEM((RNG * H + 16,), jnp.float32),
        pltpu.VMEM((RNG * H + 16,), jnp.float32),
        pltpu.VMEM((RNG * 128 + 128,), jnp.float32),
        pltpu.SemaphoreType.DMA,
    ],
)
def _sc_kernel(z_hbm, asd_hbm, src_hbm, dst_hbm, et_hbm, out_hbm, *scratch):
    _sc_body(z_hbm, asd_hbm, src_hbm, dst_hbm, et_hbm, out_hbm, *scratch)


def kernel(feature, fc_weight, attn_weight, edge_index, etype):
    A = attn_weight.reshape(R, H, 2, DH)
    eye = jnp.eye(H, dtype=jnp.float32)
    Ws = jnp.einsum('rhk,hj->rhkj', A[:, :, 0, :], eye).reshape(R, OUT_DIM, H)
    Wd = jnp.einsum('rhk,hj->rhkj', A[:, :, 1, :], eye).reshape(R, OUT_DIM, H)
    wsd = jnp.concatenate([Ws, Wd], axis=2)

    z_all, asd = pl.pallas_call(
        _tc_body,
        grid=(NBLK,),
        in_specs=[
            pl.BlockSpec((BLK, IN_DIM), lambda i: (i, 0)),
            pl.BlockSpec((R, IN_DIM, OUT_DIM), lambda i: (0, 0, 0)),
            pl.BlockSpec((R, OUT_DIM, 16), lambda i: (0, 0, 0)),
        ],
        out_specs=[
            pl.BlockSpec((BLK, R, OUT_DIM), lambda i: (i, 0, 0)),
            pl.BlockSpec((BLK, R, 16), lambda i: (i, 0, 0)),
        ],
        out_shape=[
            jax.ShapeDtypeStruct((N, R, OUT_DIM), jnp.float32),
            jax.ShapeDtypeStruct((N, R, 16), jnp.float32),
        ],
    )(feature, fc_weight, wsd)

    z_flat = z_all.reshape(N * R, OUT_DIM)
    asd_flat = asd.reshape(N * R, 16)
    src = edge_index[0]
    dst = edge_index[1]
    out1d = _sc_kernel(z_flat, asd_flat, src, dst, etype)
    return out1d[:N * 128].reshape(N, H, DH)
